# baseline (speedup 1.0000x reference)
import functools

import jax
import jax.numpy as jnp
from jax import lax
from jax.experimental import pallas as pl
from jax.experimental.pallas import tpu as pltpu
from jax.experimental.pallas import tpu_sc as plsc

N = 4096
NNZ = 1677721
NS = 16
L = 16
BLK = 32768
NBLK = 52
NNZ_PAD = NBLK * BLK
TOTAL_PAD = 3 * NNZ_PAD
W = 2048
WROWS = W // 128
C = TOTAL_PAD // NS
NFLAT = N * N
RANGE = 1441792
TW = RANGE // NS
DUMP_MASK = (1 << 20) - 1
XB = 4096


def _prep_body(r0, c0, v0, r1, c1, v1, r2, c2, v2, ar, flat_ref, vals_ref):
    op = pl.program_id(0)
    j = pl.program_id(1)
    r = jnp.where(op == 0, r0[...], jnp.where(op == 1, r1[...], r2[...]))
    c = jnp.where(op == 0, c0[...], jnp.where(op == 1, c1[...], c2[...]))
    v = jnp.where(op == 0, v0[...], jnp.where(op == 1, v1[...], v2[...]))
    p = j * BLK + ar[...]
    valid = p < NNZ
    flat_ref[...] = jnp.where(valid, (r << 12) | c, p)
    vals_ref[...] = jnp.where(valid, v, 0.0)


def _in_spec(i):
    return pl.BlockSpec((BLK,), lambda op, j, i=i: (jnp.where(op == i, j, 0),))


_tc_prep = pl.pallas_call(
    _prep_body,
    grid=(3, NBLK),
    in_specs=[s for i in range(3) for s in (_in_spec(i),) * 3]
    + [pl.BlockSpec((BLK,), lambda op, j: (0,))],
    out_specs=[
        pl.BlockSpec((BLK,), lambda op, j: (op * NBLK + j,)),
        pl.BlockSpec((BLK,), lambda op, j: (op * NBLK + j,)),
    ],
    out_shape=[
        jax.ShapeDtypeStruct((TOTAL_PAD,), jnp.int32),
        jax.ShapeDtypeStruct((TOTAL_PAD,), jnp.float32),
    ],
)


@functools.partial(
    pl.kernel,
    out_type=jax.ShapeDtypeStruct((12 * RANGE,), jnp.float32),
    mesh=plsc.VectorSubcoreMesh(core_axis_name="c", subcore_axis_name="s"),
    scratch_types=[
        pltpu.VMEM((2, W), jnp.int32),
        pltpu.VMEM((4, W), jnp.float32),
        pltpu.VMEM((4, WROWS, 128), jnp.int32),
        pltpu.VMEM((XB,), jnp.float32),
        pltpu.VMEM((2, XB), jnp.float32),
        pltpu.VMEM_SHARED((RANGE,), jnp.float32),
        pltpu.SemaphoreType.DMA,
        pltpu.SemaphoreType.DMA,
        pltpu.SemaphoreType.DMA,
        pltpu.SemaphoreType.DMA,
        pltpu.SemaphoreType.DMA,
        pltpu.SemaphoreType.DMA,
        pltpu.SemaphoreType.DMA,
        pltpu.SemaphoreType.DMA,
        pltpu.SemaphoreType.DMA,
        pltpu.SemaphoreType.DMA,
        pltpu.SemaphoreType.DMA,
        pltpu.SemaphoreType.DMA,
    ],
)
def _sc_scatter(flat_hbm, vals_hbm, zeros_hbm, out_hbm,
                flat_v, vals_v, idx_v, zv, xb, acc,
                f0, f1, v0, v1, v2, v3, s0, s1, s2, s3, sem_x, sem_o):
    sem_f = (f0, f1)
    sem_v = (v0, v1, v2, v3)
    sem_sc = (s0, s1, s2, s3)
    core = lax.axis_index("c")
    tid = lax.axis_index("s")
    chunk0 = tid * C
    pltpu.sync_copy(zeros_hbm, zv)

    def fire_in(start, b, v):
        start = pl.multiple_of(start, 128)
        pltpu.async_copy(flat_hbm.at[pl.ds(start, W)], flat_v.at[b],
                         sem_f[b])
        pltpu.async_copy(vals_hbm.at[pl.ds(start, W)], vals_v.at[v],
                         sem_v[v])

    def wait_in(b, v):
        pltpu.make_async_copy(flat_hbm.at[pl.ds(0, W)], flat_v.at[b],
                              sem_f[b]).wait()
        pltpu.make_async_copy(vals_hbm.at[pl.ds(0, W)], vals_v.at[v],
                              sem_v[v]).wait()

    def drain_scatter(v):
        pltpu.make_async_copy(vals_hbm.at[pl.ds(0, W)], vals_v.at[v],
                              sem_sc[v]).wait()

    def one_pass(range_id, tw):
        base_u = (range_id * RANGE).astype(jnp.uint32)
        @pl.loop(0, TW, step=XB)
        def _zf(k):
            pltpu.async_copy(
                zv, acc.at[pl.ds(pl.multiple_of(tid * TW + k, 8), XB)],
                sem_x)

        @pl.loop(0, TW, step=XB)
        def _zw(k):
            pltpu.make_async_copy(zeros_hbm, zv, sem_x).wait()
        plsc.subcore_barrier()

        fire_in(chunk0, 0, 0)
        fire_in(chunk0 + W, 1, 1)

        @pl.loop(0, C, step=4 * W)
        def _g(w):
            for u in range(4):
                b = u % 2
                wu = w + u * W
                wait_in(b, u)

                @pl.loop(0, WROWS)
                def _row(j):
                    for i in range(0, 128, L):
                        f = flat_v[b, pl.ds(j * 128 + i, L)]
                        rel = f.astype(jnp.uint32) - base_u
                        msk = rel < jnp.uint32(RANGE)
                        alt = rel & jnp.uint32(DUMP_MASK)
                        idx_v[u, j, pl.ds(i, L)] = jnp.where(
                            msk, rel, alt).astype(jnp.int32)
                        v = vals_v[u, pl.ds(j * 128 + i, L)]
                        vals_v[u, pl.ds(j * 128 + i, L)] = jnp.where(
                            msk, v, 0.0)

                @pl.loop(0, WROWS)
                def _fire(j):
                    pltpu.async_copy(
                        vals_v.at[u, pl.ds(pl.multiple_of(j * 128, 128), 128)],
                        acc.at[idx_v.at[u, j]], sem_sc[u], add=True)

                nv = (u + 2) % 4
                if u < 2:
                    @pl.when(w > 0)
                    def _():
                        drain_scatter(nv)
                else:
                    drain_scatter(nv)

                @pl.when(wu + 2 * W < C)
                def _():
                    fire_in(chunk0 + wu + 2 * W, b, nv)

        drain_scatter(2)
        drain_scatter(3)
        plsc.subcore_barrier()

        @pl.loop(0, tw, step=2 * XB)
        def _exp(k):
            for b in (0, 1):
                kb = k + b * XB

                @pl.when(kb >= 2 * XB)
                def _():
                    pltpu.make_async_copy(xb.at[b], out_hbm.at[pl.ds(0, XB)],
                                          sem_o).wait()

                srco = pl.multiple_of(tid * tw + kb, 8)
                pltpu.async_copy(acc.at[pl.ds(srco, XB)], xb.at[b], sem_x)
                pltpu.make_async_copy(zeros_hbm, xb.at[b], sem_x).wait()
                dst = pl.multiple_of(
                    (base_u + jnp.uint32(tid * tw + kb)).astype(jnp.int32), 8)
                pltpu.async_copy(xb.at[b], out_hbm.at[pl.ds(dst, XB)], sem_o)

        for k in range(2):
            pltpu.make_async_copy(xb.at[k], out_hbm.at[pl.ds(0, XB)],
                                  sem_o).wait()

    @pl.loop(0, 6)
    def _p(p):
        one_pass(core * 6 + p, TW)


def kernel(rows0, cols0, vals0, rows1, cols1, vals1, rows2, cols2, vals2):
    ar = jnp.arange(BLK, dtype=jnp.int32)
    flat_all, vals_all = _tc_prep(rows0, cols0, vals0, rows1, cols1, vals1,
                                  rows2, cols2, vals2, ar)
    zeros = jnp.zeros((XB,), jnp.float32)
    flat = _sc_scatter(flat_all, vals_all, zeros)
    return flat[:NFLAT].reshape(N, N)


# speedup vs baseline: 1.1344x; 1.1344x over previous
import functools

import jax
import jax.numpy as jnp
from jax import lax
from jax.experimental import pallas as pl
from jax.experimental.pallas import tpu as pltpu
from jax.experimental.pallas import tpu_sc as plsc

N = 4096
NNZ = 1677721
NS = 16
L = 16
BLK = 32768
NBLK = 52
NNZ_PAD = NBLK * BLK
TOTAL_PAD = 3 * NNZ_PAD
W = 2048
WROWS = W // 128
C = TOTAL_PAD // NS
NFLAT = N * N
RANGE = 1703936
TW = RANGE // NS
DUMP_MASK = (1 << 20) - 1
ZB = 1024
XB = 1024


def _prep_body(r0, c0, v0, r1, c1, v1, r2, c2, v2, ar, flat_ref, vals_ref):
    op = pl.program_id(0)
    j = pl.program_id(1)
    r = jnp.where(op == 0, r0[...], jnp.where(op == 1, r1[...], r2[...]))
    c = jnp.where(op == 0, c0[...], jnp.where(op == 1, c1[...], c2[...]))
    v = jnp.where(op == 0, v0[...], jnp.where(op == 1, v1[...], v2[...]))
    p = j * BLK + ar[...]
    valid = p < NNZ
    flat_ref[...] = jnp.where(valid, (r << 12) | c, p)
    vals_ref[...] = jnp.where(valid, v, 0.0)


def _in_spec(i):
    return pl.BlockSpec((BLK,), lambda op, j, i=i: (jnp.where(op == i, j, 0),))


_tc_prep = pl.pallas_call(
    _prep_body,
    grid=(3, NBLK),
    in_specs=[s for i in range(3) for s in (_in_spec(i),) * 3]
    + [pl.BlockSpec((BLK,), lambda op, j: (0,))],
    out_specs=[
        pl.BlockSpec((BLK,), lambda op, j: (op * NBLK + j,)),
        pl.BlockSpec((BLK,), lambda op, j: (op * NBLK + j,)),
    ],
    out_shape=[
        jax.ShapeDtypeStruct((TOTAL_PAD,), jnp.int32),
        jax.ShapeDtypeStruct((TOTAL_PAD,), jnp.float32),
    ],
)


@functools.partial(
    pl.kernel,
    out_type=jax.ShapeDtypeStruct((10 * RANGE,), jnp.float32),
    mesh=plsc.VectorSubcoreMesh(core_axis_name="c", subcore_axis_name="s"),
    scratch_types=[
        pltpu.VMEM((2, W), jnp.int32),
        pltpu.VMEM((4, W), jnp.float32),
        pltpu.VMEM((4, WROWS, 128), jnp.int32),
        pltpu.VMEM((ZB,), jnp.float32),
        pltpu.VMEM((2, XB), jnp.float32),
        pltpu.VMEM_SHARED((RANGE,), jnp.float32),
        pltpu.SemaphoreType.DMA,
        pltpu.SemaphoreType.DMA,
        pltpu.SemaphoreType.DMA,
        pltpu.SemaphoreType.DMA,
        pltpu.SemaphoreType.DMA,
        pltpu.SemaphoreType.DMA,
        pltpu.SemaphoreType.DMA,
        pltpu.SemaphoreType.DMA,
        pltpu.SemaphoreType.DMA,
        pltpu.SemaphoreType.DMA,
        pltpu.SemaphoreType.DMA,
        pltpu.SemaphoreType.DMA,
    ],
)
def _sc_scatter(flat_hbm, vals_hbm, zeros_hbm, out_hbm,
                flat_v, vals_v, idx_v, zv, xb, acc,
                f0, f1, v0, v1, v2, v3, s0, s1, s2, s3, sem_x, sem_o):
    sem_f = (f0, f1)
    sem_v = (v0, v1, v2, v3)
    sem_sc = (s0, s1, s2, s3)
    core = lax.axis_index("c")
    tid = lax.axis_index("s")
    chunk0 = tid * C
    pltpu.sync_copy(zeros_hbm, zv)

    def fire_in(start, b, v):
        start = pl.multiple_of(start, 128)
        pltpu.async_copy(flat_hbm.at[pl.ds(start, W)], flat_v.at[b],
                         sem_f[b])
        pltpu.async_copy(vals_hbm.at[pl.ds(start, W)], vals_v.at[v],
                         sem_v[v])

    def wait_in(b, v):
        pltpu.make_async_copy(flat_hbm.at[pl.ds(0, W)], flat_v.at[b],
                              sem_f[b]).wait()
        pltpu.make_async_copy(vals_hbm.at[pl.ds(0, W)], vals_v.at[v],
                              sem_v[v]).wait()

    def drain_scatter(v):
        pltpu.make_async_copy(vals_hbm.at[pl.ds(0, W)], vals_v.at[v],
                              sem_sc[v]).wait()

    def one_pass(range_id, tw):
        base_u = (range_id * RANGE).astype(jnp.uint32)
        @pl.loop(0, TW, step=ZB)
        def _zf(k):
            pltpu.async_copy(
                zv, acc.at[pl.ds(pl.multiple_of(tid * TW + k, 8), ZB)],
                sem_x)

        @pl.loop(0, TW, step=ZB)
        def _zw(k):
            pltpu.make_async_copy(zeros_hbm, zv, sem_x).wait()
        plsc.subcore_barrier()

        fire_in(chunk0, 0, 0)
        fire_in(chunk0 + W, 1, 1)

        @pl.loop(0, C, step=4 * W)
        def _g(w):
            for u in range(4):
                b = u % 2
                wu = w + u * W
                wait_in(b, u)

                @pl.loop(0, WROWS)
                def _row(j):
                    for i in range(0, 128, L):
                        f = flat_v[b, pl.ds(j * 128 + i, L)]
                        rel = f.astype(jnp.uint32) - base_u
                        msk = rel < jnp.uint32(RANGE)
                        alt = rel & jnp.uint32(DUMP_MASK)
                        idx_v[u, j, pl.ds(i, L)] = jnp.where(
                            msk, rel, alt).astype(jnp.int32)
                        v = vals_v[u, pl.ds(j * 128 + i, L)]
                        vals_v[u, pl.ds(j * 128 + i, L)] = jnp.where(
                            msk, v, 0.0)

                @pl.loop(0, WROWS)
                def _fire(j):
                    pltpu.async_copy(
                        vals_v.at[u, pl.ds(pl.multiple_of(j * 128, 128), 128)],
                        acc.at[idx_v.at[u, j]], sem_sc[u], add=True)

                nv = (u + 2) % 4
                if u < 2:
                    @pl.when(w > 0)
                    def _():
                        drain_scatter(nv)
                else:
                    drain_scatter(nv)

                @pl.when(wu + 2 * W < C)
                def _():
                    fire_in(chunk0 + wu + 2 * W, b, nv)

        drain_scatter(2)
        drain_scatter(3)
        plsc.subcore_barrier()

        @pl.loop(0, tw, step=2 * XB)
        def _exp(k):
            for b in (0, 1):
                kb = k + b * XB

                @pl.when(kb >= 2 * XB)
                def _():
                    pltpu.make_async_copy(xb.at[b], out_hbm.at[pl.ds(0, XB)],
                                          sem_o).wait()

                srco = pl.multiple_of(tid * tw + kb, 8)
                pltpu.async_copy(acc.at[pl.ds(srco, XB)], xb.at[b], sem_x)
                pltpu.make_async_copy(zeros_hbm, xb.at[b], sem_x).wait()
                dst = pl.multiple_of(
                    (base_u + jnp.uint32(tid * tw + kb)).astype(jnp.int32), 8)
                pltpu.async_copy(xb.at[b], out_hbm.at[pl.ds(dst, XB)], sem_o)

        for k in range(2):
            pltpu.make_async_copy(xb.at[k], out_hbm.at[pl.ds(0, XB)],
                                  sem_o).wait()

    @pl.loop(0, 5)
    def _p(p):
        one_pass(core * 5 + p, TW)


def kernel(rows0, cols0, vals0, rows1, cols1, vals1, rows2, cols2, vals2):
    ar = jnp.arange(BLK, dtype=jnp.int32)
    flat_all, vals_all = _tc_prep(rows0, cols0, vals0, rows1, cols1, vals1,
                                  rows2, cols2, vals2, ar)
    zeros = jnp.zeros((ZB,), jnp.float32)
    flat = _sc_scatter(flat_all, vals_all, zeros)
    return flat[:NFLAT].reshape(N, N)


# speedup vs baseline: 1.1693x; 1.0308x over previous
import functools

import jax
import jax.numpy as jnp
from jax import lax
from jax.experimental import pallas as pl
from jax.experimental.pallas import tpu as pltpu
from jax.experimental.pallas import tpu_sc as plsc

N = 4096
NNZ = 1677721
NS = 16
L = 16
BLK = 32768
NBLK = 52
NNZ_PAD = NBLK * BLK
TOTAL_PAD = 3 * NNZ_PAD
W = 2048
WROWS = W // 128
C = TOTAL_PAD // NS
NFLAT = N * N
RANGE = 1703936
TW = RANGE // NS
LAST_WORDS = NFLAT - 9 * RANGE
TW_LAST = LAST_WORDS // NS
DUMP_MASK = (1 << 20) - 1
ZB = 1024
XB = 1024


def _prep_body(r0, c0, v0, r1, c1, v1, r2, c2, v2, ar, flat_ref, vals_ref):
    op = pl.program_id(0)
    j = pl.program_id(1)
    r = jnp.where(op == 0, r0[...], jnp.where(op == 1, r1[...], r2[...]))
    c = jnp.where(op == 0, c0[...], jnp.where(op == 1, c1[...], c2[...]))
    v = jnp.where(op == 0, v0[...], jnp.where(op == 1, v1[...], v2[...]))
    p = j * BLK + ar[...]
    valid = p < NNZ
    flat_ref[...] = jnp.where(valid, (r << 12) | c, p)
    vals_ref[...] = jnp.where(valid, v, 0.0)


def _in_spec(i):
    return pl.BlockSpec((BLK,), lambda op, j, i=i: (jnp.where(op == i, j, 0),))


_tc_prep = pl.pallas_call(
    _prep_body,
    grid=(3, NBLK),
    in_specs=[s for i in range(3) for s in (_in_spec(i),) * 3]
    + [pl.BlockSpec((BLK,), lambda op, j: (0,))],
    out_specs=[
        pl.BlockSpec((BLK,), lambda op, j: (op * NBLK + j,)),
        pl.BlockSpec((BLK,), lambda op, j: (op * NBLK + j,)),
    ],
    out_shape=[
        jax.ShapeDtypeStruct((TOTAL_PAD,), jnp.int32),
        jax.ShapeDtypeStruct((TOTAL_PAD,), jnp.float32),
    ],
)


@functools.partial(
    pl.kernel,
    out_type=jax.ShapeDtypeStruct((NFLAT,), jnp.float32),
    mesh=plsc.VectorSubcoreMesh(core_axis_name="c", subcore_axis_name="s"),
    scratch_types=[
        pltpu.VMEM((2, W), jnp.int32),
        pltpu.VMEM((4, W), jnp.float32),
        pltpu.VMEM((4, WROWS, 128), jnp.int32),
        pltpu.VMEM((ZB,), jnp.float32),
        pltpu.VMEM((2, XB), jnp.float32),
        pltpu.VMEM_SHARED((RANGE,), jnp.float32),
        pltpu.SemaphoreType.DMA,
        pltpu.SemaphoreType.DMA,
        pltpu.SemaphoreType.DMA,
        pltpu.SemaphoreType.DMA,
        pltpu.SemaphoreType.DMA,
        pltpu.SemaphoreType.DMA,
        pltpu.SemaphoreType.DMA,
        pltpu.SemaphoreType.DMA,
        pltpu.SemaphoreType.DMA,
        pltpu.SemaphoreType.DMA,
        pltpu.SemaphoreType.DMA,
        pltpu.SemaphoreType.DMA,
    ],
)
def _sc_scatter(flat_hbm, vals_hbm, zeros_hbm, out_hbm,
                flat_v, vals_v, idx_v, zv, xb, acc,
                f0, f1, v0, v1, v2, v3, s0, s1, s2, s3, sem_x, sem_o):
    sem_f = (f0, f1)
    sem_v = (v0, v1, v2, v3)
    sem_sc = (s0, s1, s2, s3)
    core = lax.axis_index("c")
    tid = lax.axis_index("s")
    chunk0 = tid * C
    pltpu.sync_copy(zeros_hbm, zv)

    def fire_in(start, b, v):
        start = pl.multiple_of(start, 128)
        pltpu.async_copy(flat_hbm.at[pl.ds(start, W)], flat_v.at[b],
                         sem_f[b])
        pltpu.async_copy(vals_hbm.at[pl.ds(start, W)], vals_v.at[v],
                         sem_v[v])

    def wait_in(b, v):
        pltpu.make_async_copy(flat_hbm.at[pl.ds(0, W)], flat_v.at[b],
                              sem_f[b]).wait()
        pltpu.make_async_copy(vals_hbm.at[pl.ds(0, W)], vals_v.at[v],
                              sem_v[v]).wait()

    def drain_scatter(v):
        pltpu.make_async_copy(vals_hbm.at[pl.ds(0, W)], vals_v.at[v],
                              sem_sc[v]).wait()

    def one_pass(range_id):
        base_u = (range_id * RANGE).astype(jnp.uint32)
        tw = jnp.where(range_id == 9, TW_LAST, TW)
        @pl.loop(0, TW, step=ZB)
        def _zf(k):
            pltpu.async_copy(
                zv, acc.at[pl.ds(pl.multiple_of(tid * TW + k, 8), ZB)],
                sem_x)

        @pl.loop(0, TW, step=ZB)
        def _zw(k):
            pltpu.make_async_copy(zeros_hbm, zv, sem_x).wait()
        plsc.subcore_barrier()

        fire_in(chunk0, 0, 0)
        fire_in(chunk0 + W, 1, 1)

        @pl.loop(0, C, step=4 * W)
        def _g(w):
            for u in range(4):
                b = u % 2
                wu = w + u * W
                wait_in(b, u)

                @pl.loop(0, WROWS)
                def _row(j):
                    for i in range(0, 128, L):
                        f = flat_v[b, pl.ds(j * 128 + i, L)]
                        rel = f.astype(jnp.uint32) - base_u
                        msk = rel < jnp.uint32(RANGE)
                        alt = rel & jnp.uint32(DUMP_MASK)
                        idx_v[u, j, pl.ds(i, L)] = jnp.where(
                            msk, rel, alt).astype(jnp.int32)
                        v = vals_v[u, pl.ds(j * 128 + i, L)]
                        vals_v[u, pl.ds(j * 128 + i, L)] = jnp.where(
                            msk, v, 0.0)

                @pl.loop(0, WROWS)
                def _fire(j):
                    pltpu.async_copy(
                        vals_v.at[u, pl.ds(pl.multiple_of(j * 128, 128), 128)],
                        acc.at[idx_v.at[u, j]], sem_sc[u], add=True)

                nv = (u + 2) % 4
                if u < 2:
                    @pl.when(w > 0)
                    def _():
                        drain_scatter(nv)
                else:
                    drain_scatter(nv)

                @pl.when(wu + 2 * W < C)
                def _():
                    fire_in(chunk0 + wu + 2 * W, b, nv)

        drain_scatter(2)
        drain_scatter(3)
        plsc.subcore_barrier()

        @pl.loop(0, tw, step=2 * XB)
        def _exp(k):
            for b in (0, 1):
                kb = k + b * XB
                kb = pl.multiple_of(kb, 8)

                @pl.when(kb >= 2 * XB)
                def _():
                    pltpu.make_async_copy(xb.at[b], out_hbm.at[pl.ds(0, XB)],
                                          sem_o).wait()

                srco = pl.multiple_of(tid * tw + kb, 8)
                pltpu.async_copy(acc.at[pl.ds(srco, XB)], xb.at[b], sem_x)
                pltpu.make_async_copy(zeros_hbm, xb.at[b], sem_x).wait()
                dst = pl.multiple_of(
                    (base_u + jnp.uint32(tid * tw + kb)).astype(jnp.int32), 8)
                pltpu.async_copy(xb.at[b], out_hbm.at[pl.ds(dst, XB)], sem_o)

        for k in range(2):
            pltpu.make_async_copy(xb.at[k], out_hbm.at[pl.ds(0, XB)],
                                  sem_o).wait()

    @pl.loop(0, 5)
    def _p(p):
        one_pass(core * 5 + p)


def kernel(rows0, cols0, vals0, rows1, cols1, vals1, rows2, cols2, vals2):
    ar = jnp.arange(BLK, dtype=jnp.int32)
    flat_all, vals_all = _tc_prep(rows0, cols0, vals0, rows1, cols1, vals1,
                                  rows2, cols2, vals2, ar)
    zeros = jnp.zeros((ZB,), jnp.float32)
    flat = _sc_scatter(flat_all, vals_all, zeros)
    return flat.reshape(N, N)
